# Initial kernel scaffold; baseline (speedup 1.0000x reference)
import jax
import jax.numpy as jnp
from jax.experimental import pallas as pl


def kernel(x, emb_weight):
    raise NotImplementedError("write your pallas kernel here")


# speedup vs baseline: 4.8021x; 4.8021x over previous
import functools

import jax
import jax.numpy as jnp
from jax import lax
from jax.experimental import pallas as pl
from jax.experimental.pallas import tpu as pltpu
from jax.experimental.pallas import tpu_sc as plsc

_NC = 2
_NS = 16
_NW = _NC * _NS
_CH = 1024


@functools.lru_cache(maxsize=None)
def _build(n_total: int, vocab: int, dim: int):
    per_w = n_total // _NW
    n_chunks = per_w // _CH

    def body(idx_hbm, table_hbm, out_hbm, idx_v, rows_v, sem):
        wid = lax.axis_index("s") * _NC + lax.axis_index("c")
        base = wid * per_w

        @pl.loop(0, n_chunks)
        def _(i):
            off = base + i * _CH
            pltpu.sync_copy(idx_hbm.at[pl.ds(off, _CH)], idx_v)
            pltpu.async_copy(table_hbm.at[idx_v], rows_v, sem).wait()
            pltpu.sync_copy(rows_v, out_hbm.at[pl.ds(off, _CH)])

    return pl.kernel(
        body,
        out_type=jax.ShapeDtypeStruct((n_total, dim), jnp.float32),
        compiler_params=pltpu.CompilerParams(use_tc_tiling_on_sc=False),
        mesh=plsc.VectorSubcoreMesh(core_axis_name="c", subcore_axis_name="s"),
        scratch_types=[
            pltpu.VMEM((_CH,), jnp.int32),
            pltpu.VMEM((_CH, dim), jnp.float32),
            pltpu.SemaphoreType.DMA,
        ],
    )


def kernel(x, emb_weight):
    flat = x.reshape(-1).astype(jnp.int32)
    vocab, dim = emb_weight.shape
    out = _build(flat.shape[0], vocab, dim)(flat, emb_weight)
    return (out.reshape(x.shape + (dim,)), 0.0)


# speedup vs baseline: 5.0427x; 1.0501x over previous
import functools

import jax
import jax.numpy as jnp
from jax import lax
from jax.experimental import pallas as pl
from jax.experimental.pallas import tpu as pltpu
from jax.experimental.pallas import tpu_sc as plsc

_NC = 2
_NS = 16
_NW = _NC * _NS
_CH = 1024
_NBUF = 3


@functools.lru_cache(maxsize=None)
def _build(n_total: int, vocab: int, dim: int):
    per_w = n_total // _NW
    n_chunks = per_w // _CH
    assert n_chunks >= 4

    def body(idx_hbm, table_hbm, out_hbm, *scratch):
        idx_v = scratch[0:_NBUF]
        rows = scratch[_NBUF:2 * _NBUF]
        sem_g = scratch[2 * _NBUF:3 * _NBUF]
        sem_o = scratch[3 * _NBUF:4 * _NBUF]

        wid = lax.axis_index("s") * _NC + lax.axis_index("c")
        base = wid * per_w

        def idx_copy(i, b):
            pltpu.sync_copy(idx_hbm.at[pl.ds(base + i * _CH, _CH)], idx_v[b])

        def gather_start(i, b):
            pltpu.async_copy(table_hbm.at[idx_v[b]], rows[b], sem_g[b])

        def gather_wait(i, b):
            pltpu.make_async_copy(table_hbm.at[idx_v[b]], rows[b],
                                  sem_g[b]).wait()

        def out_start(i, b):
            pltpu.async_copy(rows[b],
                             out_hbm.at[pl.ds(base + i * _CH, _CH)], sem_o[b])

        def out_wait(i, b):
            pltpu.make_async_copy(rows[b],
                                  out_hbm.at[pl.ds(base + i * _CH, _CH)],
                                  sem_o[b]).wait()

        def step(i, b, *, first=False, tail=0):
            gather_wait(i, b)
            out_start(i, b)
            if not first:
                out_wait(i - 1, (b - 1) % _NBUF)
            if not tail:
                idx_copy(i + 2, (b + 2) % _NBUF)
                gather_start(i + 2, (b + 2) % _NBUF)

        idx_copy(0, 0)
        idx_copy(1, 1)
        gather_start(0, 0)
        gather_start(1, 1)

        step(0, 0, first=True)

        n_main = ((n_chunks - 3) // _NBUF) * _NBUF

        @pl.loop(1, 1 + n_main, step=_NBUF)
        def _(i):
            for db in range(_NBUF):
                step(i + db, (1 + db) % _NBUF)

        for i in range(1 + n_main, n_chunks):
            step(i, i % _NBUF, tail=(i + 2 >= n_chunks))

        out_wait(n_chunks - 1, (n_chunks - 1) % _NBUF)

    return pl.kernel(
        body,
        out_type=jax.ShapeDtypeStruct((n_total, dim), jnp.float32),
        compiler_params=pltpu.CompilerParams(use_tc_tiling_on_sc=False),
        mesh=plsc.VectorSubcoreMesh(core_axis_name="c", subcore_axis_name="s"),
        scratch_types=(
            [pltpu.VMEM((_CH,), jnp.int32) for _ in range(_NBUF)]
            + [pltpu.VMEM((_CH, dim), jnp.float32) for _ in range(_NBUF)]
            + [pltpu.SemaphoreType.DMA for _ in range(2 * _NBUF)]
        ),
    )


def kernel(x, emb_weight):
    flat = x.reshape(-1).astype(jnp.int32)
    vocab, dim = emb_weight.shape
    out = _build(flat.shape[0], vocab, dim)(flat, emb_weight)
    return (out.reshape(x.shape + (dim,)), 0.0)
